# baseline (speedup 1.0000x reference)
import functools

import jax
import jax.numpy as jnp
from jax import lax
from jax.experimental import pallas as pl
from jax.experimental.pallas import tpu as pltpu
from jax.experimental.pallas import tpu_sc as plsc

B, J, D = 16384, 26, 128
BB = 256

_mesh = plsc.VectorSubcoreMesh(core_axis_name="c", subcore_axis_name="s")


@functools.partial(
    pl.kernel,
    out_type=(jax.ShapeDtypeStruct((J, D), jnp.float32),
              jax.ShapeDtypeStruct((128,), jnp.float32)),
    mesh=_mesh,
    scratch_types=[
        pltpu.VMEM((128,), jnp.int32),
        pltpu.VMEM((J, 128), jnp.int32),
        pltpu.VMEM((128,), jnp.float32),
        pltpu.VMEM((J, 128), jnp.float32),
        pltpu.SemaphoreType.DMA,
    ],
)
def _sc_scales(idx_hbm, lw_hbm, rep_hbm, out_hbm, sig_hbm,
               idx_v, rep_v, w_v, sv_v, sem):
    cid = lax.axis_index("c")
    sid = lax.axis_index("s")

    @pl.when(jnp.logical_and(cid == 0, sid == 0))
    def _():
        pltpu.sync_copy(idx_hbm, idx_v)
        pltpu.sync_copy(rep_hbm, rep_v)
        pltpu.async_copy(lw_hbm.at[idx_v], w_v, sem).wait()
        for t in range(128 // 16):
            wv = w_v[pl.ds(16 * t, 16)]
            w_v[pl.ds(16 * t, 16)] = 1.0 / (1.0 + jnp.exp(-wv))
        pltpu.sync_copy(w_v, sig_hbm)
        descs = [
            pltpu.async_copy(sig_hbm.at[rep_v.at[q]], sv_v.at[q], sem)
            for q in range(J)
        ]
        for d in descs:
            d.wait()
        pltpu.sync_copy(sv_v, out_hbm)


def _tc_body(scale_ref, feat_ref, out_ref):
    out_ref[...] = feat_ref[...] * scale_ref[...]


_tc_mul = pl.pallas_call(
    _tc_body,
    grid=(B // BB,),
    in_specs=[
        pl.BlockSpec((1, J, D), lambda i: (0, 0, 0)),
        pl.BlockSpec((BB, J, D), lambda i: (i, 0, 0)),
    ],
    out_specs=pl.BlockSpec((BB, J, D), lambda i: (i, 0, 0)),
    out_shape=jax.ShapeDtypeStruct((B, J, D), jnp.float32),
)


def kernel(idx, feat, layerweight):
    idxp = jnp.zeros((128,), jnp.int32).at[:J].set(
        idx.reshape(J).astype(jnp.int32))
    lwp = jnp.zeros((128,), jnp.float32).at[:100].set(layerweight)
    rep = jnp.broadcast_to(jnp.arange(J, dtype=jnp.int32)[:, None], (J, 128))
    scale, _ = _sc_scales(idxp, lwp, rep)
    return _tc_mul(scale[None], feat)


# speedup vs baseline: 3.5675x; 3.5675x over previous
import functools

import jax
import jax.numpy as jnp
from jax import lax
from jax.experimental import pallas as pl
from jax.experimental.pallas import tpu as pltpu
from jax.experimental.pallas import tpu_sc as plsc

B, J, D = 16384, 26, 128
BB = 512

_mesh = plsc.VectorSubcoreMesh(core_axis_name="c", subcore_axis_name="s")


@functools.partial(
    pl.kernel,
    out_type=jax.ShapeDtypeStruct((128,), jnp.float32),
    mesh=_mesh,
    scratch_types=[
        pltpu.VMEM((128,), jnp.int32),
        pltpu.VMEM((128,), jnp.float32),
        pltpu.SemaphoreType.DMA,
    ],
)
def _sc_scales(idx_hbm, lw_hbm, sig_hbm, idx_v, w_v, sem):
    cid = lax.axis_index("c")
    sid = lax.axis_index("s")

    @pl.when(jnp.logical_and(cid == 0, sid == 0))
    def _():
        pltpu.sync_copy(idx_hbm, idx_v)
        pltpu.async_copy(lw_hbm.at[idx_v], w_v, sem).wait()
        for t in range(128 // 16):
            wv = w_v[pl.ds(16 * t, 16)]
            w_v[pl.ds(16 * t, 16)] = 1.0 / (1.0 + jnp.exp(-wv))
        pltpu.sync_copy(w_v, sig_hbm)


def _tc_body(sig_ref, feat_ref, out_ref):
    for j in range(J):
        out_ref[j] = feat_ref[j] * sig_ref[j]


_tc_mul = pl.pallas_call(
    _tc_body,
    grid=(B // BB,),
    in_specs=[
        pl.BlockSpec(memory_space=pltpu.SMEM),
        pl.BlockSpec((J, BB, D), lambda i: (0, i, 0)),
    ],
    out_specs=pl.BlockSpec((J, BB, D), lambda i: (0, i, 0)),
    out_shape=jax.ShapeDtypeStruct((J, B, D), jnp.float32),
)


def kernel(idx, feat, layerweight):
    idxp = jnp.zeros((128,), jnp.int32).at[:J].set(
        idx.reshape(J).astype(jnp.int32))
    lwp = jnp.zeros((128,), jnp.float32).at[:100].set(layerweight)
    sig = _sc_scales(idxp, lwp)
    out_t = _tc_mul(sig, jnp.transpose(feat, (1, 0, 2)))
    return jnp.transpose(out_t, (1, 0, 2))


# speedup vs baseline: 3.9623x; 1.1107x over previous
import functools

import jax
import jax.numpy as jnp
from jax import lax
from jax.experimental import pallas as pl
from jax.experimental.pallas import tpu as pltpu

B, J, D = 16384, 26, 128
BB = 512


def _t_body(idx_ref, lw_ref, out_ref):
    k = lax.broadcasted_iota(jnp.int32, (128, 128), 0)
    idxb = jnp.broadcast_to(idx_ref[...], (128, 128))
    lwb = jnp.broadcast_to(lw_ref[...], (128, 128))
    w = jnp.sum(jnp.where(k == idxb, lwb, 0.0), axis=0, keepdims=True)
    out_ref[...] = 1.0 / (1.0 + jnp.exp(-w))


_t_scales = pl.pallas_call(
    _t_body,
    out_shape=jax.ShapeDtypeStruct((1, 128), jnp.float32),
)


def _tc_body(sig_ref, feat_ref, out_ref):
    for j in range(J):
        out_ref[j] = feat_ref[j] * sig_ref[j]


_tc_mul = pl.pallas_call(
    _tc_body,
    grid=(B // BB,),
    in_specs=[
        pl.BlockSpec(memory_space=pltpu.SMEM),
        pl.BlockSpec((J, BB, D), lambda i: (0, i, 0)),
    ],
    out_specs=pl.BlockSpec((J, BB, D), lambda i: (0, i, 0)),
    out_shape=jax.ShapeDtypeStruct((J, B, D), jnp.float32),
)


def kernel(idx, feat, layerweight):
    idxp = jnp.zeros((1, 128), jnp.int32).at[0, :J].set(
        idx.reshape(J).astype(jnp.int32))
    lwp = jnp.zeros((128, 1), jnp.float32).at[:100, 0].set(layerweight)
    sig = _t_scales(idxp, lwp)
    out_t = _tc_mul(sig.reshape(128), jnp.transpose(feat, (1, 0, 2)))
    return jnp.transpose(out_t, (1, 0, 2))
